# Initial kernel scaffold; baseline (speedup 1.0000x reference)
import jax
import jax.numpy as jnp
from jax.experimental import pallas as pl


def kernel(x, Lambda, edge_index, batch, We1, be1, We2, be2, We3, be3, Wr, br):
    raise NotImplementedError("write your pallas kernel here")


# speedup vs baseline: 11.1482x; 11.1482x over previous
import functools

import jax
import jax.numpy as jnp
from jax import lax
from jax.experimental import pallas as pl
from jax.experimental.pallas import tpu as pltpu
from jax.experimental.pallas import tpu_sc as plsc

N = 50000
E = 800000
PE = 8
B = 64
EV = 8
OUT = 16

NC = 2
NS = 16
NW = NC * NS
CH = 128
CPW = 196
EW = CPW * CH
EPAD = NW * EW


def _mtable_body(lam_ref, we1_ref, be1_ref, we2_ref, be2_ref, we3_ref,
                 be3_ref, wr8_ref, m_ref):
    lam = lam_ref[:, :]
    h = jnp.maximum(lam * we1_ref[:, :] + be1_ref[:, :], 0.0)
    h = jnp.maximum(
        jnp.dot(h, we2_ref[:, :], preferred_element_type=jnp.float32)
        + be2_ref[:, :], 0.0)
    lamf = (jnp.dot(h, we3_ref[:, :], preferred_element_type=jnp.float32)
            + be3_ref[:, :])
    m_ref[:, :] = jnp.dot(lamf, wr8_ref[:, :],
                          preferred_element_type=jnp.float32)


def _mtable(Lambda, We1, be1, We2, be2, We3, be3, Wr):
    lam_col = Lambda.reshape(B * PE, 1)
    wr8 = Wr[:EV, :]
    return pl.pallas_call(
        _mtable_body,
        out_shape=jax.ShapeDtypeStruct((B * PE, OUT), jnp.float32),
    )(lam_col, We1, be1.reshape(1, 32), We2, be2.reshape(1, 32),
      We3, be3.reshape(1, EV), wr8)


def _sc_body(src_h, dst_h, xb_h, m_h, br_h, out_h,
             src_v, dst_v, xs_v, xd_v, out_v, m_v, br_v, sem1, sem2):
    wid = lax.axis_index("s") * NC + lax.axis_index("c")
    pltpu.sync_copy(m_h, m_v)
    pltpu.sync_copy(br_h, br_v)
    brv = br_v[:]
    base0 = wid * EW

    def chunk_body(g, carry):
        base = pl.multiple_of(base0 + g * CH, CH)
        pltpu.sync_copy(src_h.at[pl.ds(base, CH)], src_v)
        pltpu.sync_copy(dst_h.at[pl.ds(base, CH)], dst_v)
        cp1 = pltpu.async_copy(xb_h.at[src_v], xs_v, sem1)
        cp2 = pltpu.async_copy(xb_h.at[dst_v], xd_v, sem2)
        cp1.wait()
        cp2.wait()

        def edge_body(e, c):
            xrow = xs_v[e, :]
            xdrow = xd_v[e, :]
            prod = xrow * xdrow
            mi = xrow[PE].astype(jnp.int32)
            acc = brv
            for pe in range(PE):
                acc = acc + prod[pe] * m_v[mi + pe]
            out_v[e, :] = acc
            return c

        lax.fori_loop(0, CH, edge_body, 0, unroll=2)
        pltpu.sync_copy(out_v, out_h.at[pl.ds(base, CH)])
        return carry

    lax.fori_loop(0, CPW, chunk_body, 0)


_sc_kernel = functools.partial(
    pl.kernel,
    out_type=jax.ShapeDtypeStruct((EPAD, OUT), jnp.float32),
    mesh=plsc.VectorSubcoreMesh(core_axis_name="c", subcore_axis_name="s",
                                num_cores=NC, num_subcores=NS),
    compiler_params=pltpu.CompilerParams(use_tc_tiling_on_sc=False),
    scratch_types=[
        pltpu.VMEM((CH,), jnp.int32),
        pltpu.VMEM((CH,), jnp.int32),
        pltpu.VMEM((CH, 16), jnp.float32),
        pltpu.VMEM((CH, 16), jnp.float32),
        pltpu.VMEM((CH, OUT), jnp.float32),
        pltpu.VMEM((B * PE, OUT), jnp.float32),
        pltpu.VMEM((16,), jnp.float32),
        pltpu.SemaphoreType.DMA,
        pltpu.SemaphoreType.DMA,
    ],
)(_sc_body)


def kernel(x, Lambda, edge_index, batch, We1, be1, We2, be2, We3, be3, Wr, br):
    m = _mtable(Lambda, We1, be1, We2, be2, We3, be3, Wr)
    xb = jnp.concatenate(
        [x, (batch * PE).astype(jnp.float32)[:, None],
         jnp.zeros((N, 16 - PE - 1), jnp.float32)], axis=1)
    src = jnp.pad(edge_index[0], (0, EPAD - E))
    dst = jnp.pad(edge_index[1], (0, EPAD - E))
    out = _sc_kernel(src, dst, xb, m, br)
    return out[:E]
